# baseline (speedup 1.0000x reference)
import functools

import jax
import jax.numpy as jnp
from jax import lax
from jax.experimental import pallas as pl
from jax.experimental.pallas import tpu as pltpu
from jax.experimental.pallas import tpu_sc as plsc

N_PROTEINS = 20000
N_RUNS = 100
BATCH = 16384
PAD_W = 128

NC = 2
NS = 16
L = 16
NW = NC * NS
BPW = BATCH // NW
BLK = 64
NBLK = BPW // BLK
N_FULL = N_RUNS // L
TAIL_BASE = N_RUNS - L


def _vmem_set():
    return [
        pltpu.VMEM((BLK, N_RUNS + 1), jnp.float32),
        pltpu.VMEM((BLK + L,), jnp.float32),
        pltpu.VMEM((BLK,), jnp.int32),
        pltpu.VMEM((BLK, PAD_W), jnp.float32),
    ]


@functools.partial(
    pl.kernel,
    out_type=jax.ShapeDtypeStruct((NW * 2 * L,), jnp.float32),
    mesh=plsc.VectorSubcoreMesh(core_axis_name="c", subcore_axis_name="s"),
    scratch_types=[
        *_vmem_set(),
        *_vmem_set(),
        pltpu.VMEM((2 * L,), jnp.float32),
        pltpu.SemaphoreType.DMA,
        pltpu.SemaphoreType.DMA,
        pltpu.SemaphoreType.DMA,
        pltpu.SemaphoreType.DMA,
    ],
)
def _sc_partial_loss(yt_hbm, yp_hbm, lab_hbm, al_hbm, out_hbm,
                     yt0, yp0, idx0, rows0, yt1, yp1, idx1, rows1,
                     part_v, sem_in0, sem_in1, sem_g0, sem_g1):
    wid = lax.axis_index("s") * NC + lax.axis_index("c")
    base = wid * BPW

    iota16 = lax.iota(jnp.int32, L)
    bufs = ((yt0, yp0, idx0, rows0, sem_in0, sem_g0),
            (yt1, yp1, idx1, rows1, sem_in1, sem_g1))

    def stage_in(b):
        yt_v, yp_v, idx_v, _, s, _ = bufs[b % 2]
        g = base + b * BLK
        return [
            pltpu.async_copy(yt_hbm.at[pl.ds(g, BLK)], yt_v, s),
            pltpu.async_copy(yp_hbm.at[pl.ds(g, BLK)],
                             yp_v.at[pl.ds(0, BLK)], s),
            pltpu.async_copy(lab_hbm.at[pl.ds(g, BLK)], idx_v, s),
        ]

    def start_gather(b):
        _, _, idx_v, rows_v, _, sg = bufs[b % 2]
        return pltpu.async_copy(al_hbm.at[idx_v], rows_v, sg)

    loss_tot = jnp.zeros((L,), jnp.float32)
    cnt_tot = jnp.zeros((L,), jnp.float32)

    for cp in stage_in(0):
        cp.wait()
    gth = [start_gather(0)]
    pend = stage_in(1)

    for b in range(NBLK):
        yt_v, yp_v, _, rows_v, _, _ = bufs[b % 2]
        if b + 1 < NBLK:
            for cp in pend:
                cp.wait()
            gth.append(start_gather(b + 1))
        gth[b].wait()

        def row_body(r, carry, yt_v=yt_v, yp_v=yp_v, rows_v=rows_v):
            loss_acc, cnt_acc = carry
            r16 = pl.multiple_of((r // L) * L, L)
            cvec = jnp.abs(yp_v[pl.ds(r16, L)])
            c = cvec.at[jnp.full((L,), r - r16, jnp.int32)].get(
                mode="promise_in_bounds")
            cnzf = jnp.where(c != 0.0, 1.0, 0.0)
            tailf = jnp.where(iota16 >= N_FULL * L - TAIL_BASE, 1.0, 0.0)
            for j in range(N_FULL + 1):
                masked = j == N_FULL
                col0 = TAIL_BASE if masked else j * L
                q = yt_v[r, pl.ds(col0, L)]
                a = rows_v[r, pl.ds(col0, L)]
                validf = jnp.where(q != 0.0, cnzf, 0.0)
                if masked:
                    validf = validf * tailf
                d = (q - c * jnp.abs(a)) * validf
                loss_acc = loss_acc + d * d
                cnt_acc = cnt_acc + validf
            return loss_acc, cnt_acc

        loss_tot, cnt_tot = lax.fori_loop(
            0, BLK, row_body, (loss_tot, cnt_tot))

        if b + 2 < NBLK:
            pend = stage_in(b + 2)

    part_v[pl.ds(0, L)] = loss_tot
    part_v[pl.ds(L, L)] = cnt_tot
    pltpu.sync_copy(part_v, out_hbm.at[pl.ds(wid * 2 * L, 2 * L)])


def kernel(y_true, y_pred, alphas):
    eye_pad = jnp.eye(N_RUNS, PAD_W, dtype=jnp.float32)
    al_pad = jnp.dot(alphas, eye_pad,
                     precision=lax.DotAlgorithmPreset.BF16_BF16_F32_X3)
    onehot = jnp.zeros((N_RUNS + 1,), jnp.float32).at[N_RUNS].set(1.0)
    labels = (jnp.dot(y_true, onehot,
                      precision=lax.DotAlgorithmPreset.BF16_BF16_F32_X3)
              + 0.5).astype(jnp.int32)
    parts = _sc_partial_loss(y_true, jnp.reshape(y_pred, (-1,)), labels,
                             al_pad)
    pr = jnp.reshape(parts, (NW, 2, L))
    total_loss = jnp.sum(pr[:, 0, :])
    all_runs = jnp.sum(pr[:, 1, :])
    return total_loss / all_runs


# speedup vs baseline: 1.0835x; 1.0835x over previous
import functools

import jax
import jax.numpy as jnp
from jax import lax
from jax.experimental import pallas as pl
from jax.experimental.pallas import tpu as pltpu
from jax.experimental.pallas import tpu_sc as plsc

N_PROTEINS = 20000
N_RUNS = 100
BATCH = 16384
PAD_W = 128

NC = 2
NS = 16
L = 16
NW = NC * NS
BPW = BATCH // NW
BLK = 128
NBLK = BPW // BLK
N_FULL = N_RUNS // L
TAIL_BASE = N_RUNS - L


def _vmem_set():
    return [
        pltpu.VMEM((BLK, N_RUNS + 1), jnp.float32),
        pltpu.VMEM((BLK + L,), jnp.float32),
        pltpu.VMEM((BLK,), jnp.int32),
        pltpu.VMEM((BLK, PAD_W), jnp.float32),
    ]


@functools.partial(
    pl.kernel,
    out_type=jax.ShapeDtypeStruct((NW * 2 * L,), jnp.float32),
    mesh=plsc.VectorSubcoreMesh(core_axis_name="c", subcore_axis_name="s"),
    scratch_types=[
        *_vmem_set(),
        *_vmem_set(),
        *_vmem_set(),
        pltpu.VMEM((2 * L,), jnp.float32),
        pltpu.SemaphoreType.DMA,
        pltpu.SemaphoreType.DMA,
        pltpu.SemaphoreType.DMA,
        pltpu.SemaphoreType.DMA,
        pltpu.SemaphoreType.DMA,
        pltpu.SemaphoreType.DMA,
    ],
)
def _sc_partial_loss(yt_hbm, yp_hbm, lab_hbm, al_hbm, out_hbm,
                     yt0, yp0, idx0, rows0, yt1, yp1, idx1, rows1,
                     yt2, yp2, idx2, rows2,
                     part_v, sem_in0, sem_in1, sem_in2,
                     sem_g0, sem_g1, sem_g2):
    wid = lax.axis_index("s") * NC + lax.axis_index("c")
    base = wid * BPW

    iota16 = lax.iota(jnp.int32, L)
    bufs = ((yt0, yp0, idx0, rows0, sem_in0, sem_g0),
            (yt1, yp1, idx1, rows1, sem_in1, sem_g1),
            (yt2, yp2, idx2, rows2, sem_in2, sem_g2))

    def stage_in(b):
        yt_v, yp_v, idx_v, _, s, _ = bufs[b % 3]
        g = base + b * BLK
        return [
            pltpu.async_copy(yt_hbm.at[pl.ds(g, BLK)], yt_v, s),
            pltpu.async_copy(yp_hbm.at[pl.ds(g, BLK)],
                             yp_v.at[pl.ds(0, BLK)], s),
            pltpu.async_copy(lab_hbm.at[pl.ds(g, BLK)], idx_v, s),
        ]

    def start_gather(b):
        _, _, idx_v, rows_v, _, sg = bufs[b % 3]
        return pltpu.async_copy(al_hbm.at[idx_v], rows_v, sg)

    loss_tot = jnp.zeros((L,), jnp.float32)
    cnt_tot = jnp.zeros((L,), jnp.float32)

    for cp in stage_in(0):
        cp.wait()
    gth = [start_gather(0)]
    pend = stage_in(1)

    for b in range(NBLK):
        yt_v, yp_v, _, rows_v, _, _ = bufs[b % 3]
        if b + 1 < NBLK:
            for cp in pend:
                cp.wait()
            gth.append(start_gather(b + 1))
        if b + 2 < NBLK:
            pend = stage_in(b + 2)
        gth[b].wait()

        def row_body(r, carry, yt_v=yt_v, yp_v=yp_v, rows_v=rows_v):
            loss_acc, cnt_acc = carry
            r16 = pl.multiple_of((r // L) * L, L)
            cvec = jnp.abs(yp_v[pl.ds(r16, L)])
            c = cvec.at[jnp.full((L,), r - r16, jnp.int32)].get(
                mode="promise_in_bounds")
            cnzf = jnp.where(c != 0.0, 1.0, 0.0)
            tailf = jnp.where(iota16 >= N_FULL * L - TAIL_BASE, 1.0, 0.0)
            for j in range(N_FULL + 1):
                masked = j == N_FULL
                col0 = TAIL_BASE if masked else j * L
                q = yt_v[r, pl.ds(col0, L)]
                a = rows_v[r, pl.ds(col0, L)]
                validf = jnp.where(q != 0.0, cnzf, 0.0)
                if masked:
                    validf = validf * tailf
                d = (q - c * jnp.abs(a)) * validf
                loss_acc = loss_acc + d * d
                cnt_acc = cnt_acc + validf
            return loss_acc, cnt_acc

        loss_tot, cnt_tot = lax.fori_loop(
            0, BLK, row_body, (loss_tot, cnt_tot))

    part_v[pl.ds(0, L)] = loss_tot
    part_v[pl.ds(L, L)] = cnt_tot
    pltpu.sync_copy(part_v, out_hbm.at[pl.ds(wid * 2 * L, 2 * L)])


def kernel(y_true, y_pred, alphas):
    eye_pad = jnp.eye(N_RUNS, PAD_W, dtype=jnp.float32)
    al_pad = jnp.dot(alphas, eye_pad,
                     precision=lax.DotAlgorithmPreset.BF16_BF16_F32_X3)
    onehot = jnp.zeros((N_RUNS + 1,), jnp.float32).at[N_RUNS].set(1.0)
    labels = (jnp.dot(y_true, onehot,
                      precision=lax.DotAlgorithmPreset.BF16_BF16_F32_X3)
              + 0.5).astype(jnp.int32)
    parts = _sc_partial_loss(y_true, jnp.reshape(y_pred, (-1,)), labels,
                             al_pad)
    pr = jnp.reshape(parts, (NW, 2, L))
    total_loss = jnp.sum(pr[:, 0, :])
    all_runs = jnp.sum(pr[:, 1, :])
    return total_loss / all_runs
